# baseline (speedup 1.0000x reference)
import functools

import jax
import jax.numpy as jnp
from jax import lax
from jax.experimental import pallas as pl
from jax.experimental.pallas import tpu as pltpu
from jax.experimental.pallas import tpu_sc as plsc

D_PROJ = 128
CUTS = (0, 20000, 100000, 500000, 1000000)
D_EMBS = (128, 64, 32, 16)

BLK = 16384
SEG_BLKS = tuple(-(-(CUTS[i + 1] - CUTS[i]) // BLK) for i in range(4))
SEG_START = (0,
             SEG_BLKS[0],
             SEG_BLKS[0] + SEG_BLKS[1],
             SEG_BLKS[0] + SEG_BLKS[1] + SEG_BLKS[2])
N_BLOCKS = sum(SEG_BLKS)
TBL_ROWS = N_BLOCKS * BLK
OFFS = tuple(SEG_START[c] * BLK - CUTS[c] for c in range(4))


def _build_body(e0, e1t, e2t, e3t, p0, p1, p2, p3, out):
    g = pl.program_id(0)

    def mm_n(e, p):
        out[...] = lax.dot_general(
            e[...], p[...], (((1,), (1,)), ((), ())),
            preferred_element_type=jnp.float32)

    def mm_t(et, p):
        out[...] = lax.dot_general(
            et[...], p[...], (((0,), (1,)), ((), ())),
            preferred_element_type=jnp.float32)

    pl.when(g < SEG_START[1])(lambda: mm_n(e0, p0))
    pl.when((g >= SEG_START[1]) & (g < SEG_START[2]))(lambda: mm_t(e1t, p1))
    pl.when((g >= SEG_START[2]) & (g < SEG_START[3]))(lambda: mm_t(e2t, p2))
    pl.when(g >= SEG_START[3])(lambda: mm_t(e3t, p3))


def _build_table(e0, e1t, e2t, e3t, p0, p1, p2, p3):
    emb_specs = [
        pl.BlockSpec((BLK, D_EMBS[0]),
                     lambda g: (jnp.clip(g, 0, SEG_BLKS[0] - 1), 0)),
    ] + [
        pl.BlockSpec((D_EMBS[i], BLK),
                     functools.partial(
                         lambda i, g: (0, jnp.clip(g - SEG_START[i], 0,
                                                   SEG_BLKS[i] - 1)), i))
        for i in (1, 2, 3)
    ]
    proj_specs = [
        pl.BlockSpec((D_PROJ, D_EMBS[i]), lambda g: (0, 0)) for i in range(4)
    ]
    return pl.pallas_call(
        _build_body,
        grid=(N_BLOCKS,),
        in_specs=emb_specs + proj_specs,
        out_specs=pl.BlockSpec((BLK, D_PROJ), lambda g: (g, 0)),
        out_shape=jax.ShapeDtypeStruct((TBL_ROWS, D_PROJ), jnp.float32),
    )(e0, e1t, e2t, e3t, p0, p1, p2, p3)


NC, NS = 2, 16
NW = NC * NS
N_TOK = 4096 * 200
BPW = N_TOK // NW
CH = 160
NBUF = 4
ITERS = BPW // CH


@functools.lru_cache(maxsize=1)
def _make_gather():
    mesh = plsc.VectorSubcoreMesh(core_axis_name="c", subcore_axis_name="s")

    @functools.partial(
        pl.kernel, mesh=mesh,
        out_type=jax.ShapeDtypeStruct((N_TOK, D_PROJ), jnp.float32),
        scratch_types=(
            [pltpu.VMEM((CH,), jnp.int32) for _ in range(NBUF)]
            + [pltpu.VMEM((CH, D_PROJ), jnp.float32) for _ in range(NBUF)]
            + [pltpu.SemaphoreType.DMA for _ in range(2 * NBUF)]
        ),
    )
    def _gather(table_hbm, idx_hbm, out_hbm, *bufs):
        idx_r = bufs[:NBUF]
        rows_r = bufs[NBUF:2 * NBUF]
        semg = bufs[2 * NBUF:3 * NBUF]
        sems = bufs[3 * NBUF:]
        wid = lax.axis_index("s") * NC + lax.axis_index("c")
        base = wid * BPW

        def load_remap(j, idx_v):
            pltpu.sync_copy(idx_hbm.at[pl.ds(base + j * CH, CH)], idx_v)

            def remap(k, c):
                v = idx_v[pl.ds(k * 16, 16)]
                a = jnp.where(v >= CUTS[1], OFFS[1], 0)
                a = jnp.where(v >= CUTS[2], OFFS[2], a)
                a = jnp.where(v >= CUTS[3], OFFS[3], a)
                idx_v[pl.ds(k * 16, 16)] = v + a
                return c

            lax.fori_loop(0, CH // 16, remap, 0)

        def phase(j, s):
            nxt = (s + NBUF - 1) % NBUF

            @pl.when(j + NBUF - 1 < ITERS)
            def _():
                load_remap(j + NBUF - 1, idx_r[nxt])

            pltpu.make_async_copy(
                table_hbm.at[pl.ds(0, CH)], rows_r[s], semg[s]).wait()

            @pl.when(j >= 1)
            def _():
                pltpu.make_async_copy(
                    rows_r[nxt], out_hbm.at[pl.ds(base, CH)],
                    sems[nxt]).wait()

            @pl.when(j + NBUF - 1 < ITERS)
            def _():
                pltpu.async_copy(table_hbm.at[idx_r[nxt]], rows_r[nxt],
                                 semg[nxt])

            pltpu.async_copy(rows_r[s], out_hbm.at[pl.ds(base + j * CH, CH)],
                             sems[s])

        for c in range(NBUF - 1):
            load_remap(c, idx_r[c])
            pltpu.async_copy(table_hbm.at[idx_r[c]], rows_r[c], semg[c])

        def ring(jj, carry):
            for s in range(NBUF):
                phase(jj * NBUF + s, s)
            return carry

        lax.fori_loop(0, ITERS // NBUF, ring, 0)
        pltpu.make_async_copy(
            rows_r[(ITERS - 1) % NBUF], out_hbm.at[pl.ds(base, CH)],
            sems[(ITERS - 1) % NBUF]).wait()

    return _gather


def kernel(inp, emb_w0, emb_w1, emb_w2, emb_w3,
           proj_w0, proj_w1, proj_w2, proj_w3):
    table = _build_table(emb_w0, emb_w1.T, emb_w2.T, emb_w3.T,
                         proj_w0, proj_w1, proj_w2, proj_w3)
    flat = inp.reshape(-1).astype(jnp.int32)
    ztable = jnp.zeros((TBL_ROWS, D_PROJ), jnp.float32)
    out = _make_gather()(ztable, flat)
    return (out.reshape(inp.shape + (D_PROJ,)), table)


# speedup vs baseline: 1.2912x; 1.2912x over previous
import functools

import jax
import jax.numpy as jnp
from jax import lax
from jax.experimental import pallas as pl
from jax.experimental.pallas import tpu as pltpu
from jax.experimental.pallas import tpu_sc as plsc

D_PROJ = 128
CUTS = (0, 20000, 100000, 500000, 1000000)
D_EMBS = (128, 64, 32, 16)

BLK = 16384
SEG_BLKS = tuple(-(-(CUTS[i + 1] - CUTS[i]) // BLK) for i in range(4))
SEG_START = (0,
             SEG_BLKS[0],
             SEG_BLKS[0] + SEG_BLKS[1],
             SEG_BLKS[0] + SEG_BLKS[1] + SEG_BLKS[2])
N_BLOCKS = sum(SEG_BLKS)
TBL_ROWS = N_BLOCKS * BLK
OFFS = tuple(SEG_START[c] * BLK - CUTS[c] for c in range(4))


def _build_body(e0, e1t, e2t, e3t, p0, p1, p2, p3, out):
    g = pl.program_id(0)

    def mm_n(e, p):
        out[...] = lax.dot_general(
            e[...], p[...], (((1,), (1,)), ((), ())),
            preferred_element_type=jnp.float32)

    def mm_t(et, p):
        out[...] = lax.dot_general(
            et[...], p[...], (((0,), (1,)), ((), ())),
            preferred_element_type=jnp.float32)

    pl.when(g < SEG_START[1])(lambda: mm_n(e0, p0))
    pl.when((g >= SEG_START[1]) & (g < SEG_START[2]))(lambda: mm_t(e1t, p1))
    pl.when((g >= SEG_START[2]) & (g < SEG_START[3]))(lambda: mm_t(e2t, p2))
    pl.when(g >= SEG_START[3])(lambda: mm_t(e3t, p3))


def _build_table(e0, e1t, e2t, e3t, p0, p1, p2, p3):
    emb_specs = [
        pl.BlockSpec((BLK, D_EMBS[0]),
                     lambda g: (jnp.clip(g, 0, SEG_BLKS[0] - 1), 0)),
    ] + [
        pl.BlockSpec((D_EMBS[i], BLK),
                     functools.partial(
                         lambda i, g: (0, jnp.clip(g - SEG_START[i], 0,
                                                   SEG_BLKS[i] - 1)), i))
        for i in (1, 2, 3)
    ]
    proj_specs = [
        pl.BlockSpec((D_PROJ, D_EMBS[i]), lambda g: (0, 0)) for i in range(4)
    ]
    return pl.pallas_call(
        _build_body,
        grid=(N_BLOCKS,),
        in_specs=emb_specs + proj_specs,
        out_specs=pl.BlockSpec((BLK, D_PROJ), lambda g: (g, 0)),
        out_shape=jax.ShapeDtypeStruct((TBL_ROWS, D_PROJ), jnp.float32),
    )(e0, e1t, e2t, e3t, p0, p1, p2, p3)


NC, NS = 2, 16
NW = NC * NS
N_TOK = 4096 * 200
BPW = N_TOK // NW
CH = 320
NBUF = 3
ITERS = BPW // CH


@functools.lru_cache(maxsize=1)
def _make_gather():
    mesh = plsc.VectorSubcoreMesh(core_axis_name="c", subcore_axis_name="s")

    @functools.partial(
        pl.kernel, mesh=mesh,
        out_type=jax.ShapeDtypeStruct((N_TOK, D_PROJ), jnp.float32),
        scratch_types=(
            [pltpu.VMEM((CH,), jnp.int32) for _ in range(NBUF)]
            + [pltpu.VMEM((CH, D_PROJ), jnp.float32) for _ in range(NBUF)]
            + [pltpu.SemaphoreType.DMA for _ in range(2 * NBUF)]
        ),
    )
    def _gather(table_hbm, idx_hbm, out_hbm, *bufs):
        idx_r = bufs[:NBUF]
        rows_r = bufs[NBUF:2 * NBUF]
        semg = bufs[2 * NBUF:3 * NBUF]
        sems = bufs[3 * NBUF:]
        wid = lax.axis_index("s") * NC + lax.axis_index("c")
        base = wid * BPW

        def load_remap(j, idx_v):
            pltpu.sync_copy(idx_hbm.at[pl.ds(base + j * CH, CH)], idx_v)

            def remap(k, c):
                v = idx_v[pl.ds(k * 16, 16)]
                a = jnp.where(v >= CUTS[1], OFFS[1], 0)
                a = jnp.where(v >= CUTS[2], OFFS[2], a)
                a = jnp.where(v >= CUTS[3], OFFS[3], a)
                idx_v[pl.ds(k * 16, 16)] = v + a
                return c

            lax.fori_loop(0, CH // 16, remap, 0)

        def phase(j, s):
            nxt = (s + NBUF - 1) % NBUF

            @pl.when(j + NBUF - 1 < ITERS)
            def _():
                load_remap(j + NBUF - 1, idx_r[nxt])

            pltpu.make_async_copy(
                table_hbm.at[pl.ds(0, CH)], rows_r[s], semg[s]).wait()

            @pl.when(j >= 1)
            def _():
                pltpu.make_async_copy(
                    rows_r[nxt], out_hbm.at[pl.ds(base, CH)],
                    sems[nxt]).wait()

            @pl.when(j + NBUF - 1 < ITERS)
            def _():
                pltpu.async_copy(table_hbm.at[idx_r[nxt]], rows_r[nxt],
                                 semg[nxt])

            pltpu.async_copy(rows_r[s], out_hbm.at[pl.ds(base + j * CH, CH)],
                             sems[s])

        for c in range(NBUF - 1):
            load_remap(c, idx_r[c])
            pltpu.async_copy(table_hbm.at[idx_r[c]], rows_r[c], semg[c])

        def ring(jj, carry):
            for s in range(NBUF):
                phase(jj * NBUF + s, s)
            return carry

        lax.fori_loop(0, ITERS // NBUF, ring, 0)
        for s in range(ITERS % NBUF):
            phase(ITERS - ITERS % NBUF + s, s)
        pltpu.make_async_copy(
            rows_r[(ITERS - 1) % NBUF], out_hbm.at[pl.ds(base, CH)],
            sems[(ITERS - 1) % NBUF]).wait()

    return _gather


def kernel(inp, emb_w0, emb_w1, emb_w2, emb_w3,
           proj_w0, proj_w1, proj_w2, proj_w3):
    table = _build_table(emb_w0, emb_w1.T, emb_w2.T, emb_w3.T,
                         proj_w0, proj_w1, proj_w2, proj_w3)
    flat = inp.reshape(-1).astype(jnp.int32)
    out = _make_gather()(table, flat)
    return out.reshape(inp.shape + (D_PROJ,))
